# Initial kernel scaffold; baseline (speedup 1.0000x reference)
import jax
import jax.numpy as jnp
from jax.experimental import pallas as pl


def kernel(x, edge_index, W1, b1, W2, b2, Wl, bl):
    raise NotImplementedError("write your pallas kernel here")


# speedup vs baseline: 18.5994x; 18.5994x over previous
import functools

import jax
import jax.numpy as jnp
from jax import lax
from jax.experimental import pallas as pl
from jax.experimental.pallas import tpu as pltpu
from jax.experimental.pallas import tpu_sc as plsc

N = 10000
E = 320000
D = 128
OUT = 40

NC = 2
NS = 16
NW = NC * NS
EPW = E // NW
CH = 80
NCHUNK = EPW // CH
N_PAD = 10240
RPT = N_PAD // NS
DEG_W = 16

_mesh = plsc.VectorSubcoreMesh(core_axis_name="c", subcore_axis_name="s")


@functools.partial(
    pl.kernel,
    out_type=jax.ShapeDtypeStruct((NC, N_PAD, DEG_W), jnp.float32),
    mesh=_mesh,
    scratch_types=[
        pltpu.VMEM((NCHUNK, CH), jnp.int32),
        pltpu.VMEM((CH, DEG_W), jnp.float32),
        pltpu.VMEM_SHARED((N_PAD, DEG_W), jnp.float32),
    ],
)
def _deg_kernel(dst_hbm, out_hbm, dst_v, ones_v, acc):
    cid = lax.axis_index("c")
    sid = lax.axis_index("s")
    w = cid * NS + sid

    zeros16 = jnp.zeros((16,), jnp.float32)
    ones16 = jnp.ones((16,), jnp.float32)

    def fill(i, _):
        ones_v[i, :] = ones16
        return 0

    lax.fori_loop(0, CH, fill, 0)
    pltpu.sync_copy(dst_hbm.at[w], dst_v)

    def zrow(i, _):
        ones_v[i, :] = zeros16
        return 0

    lax.fori_loop(0, CH, zrow, 0)
    for k in range(RPT // CH):
        pltpu.sync_copy(ones_v, acc.at[pl.ds(sid * RPT + k * CH, CH)])
    lax.fori_loop(0, CH, fill, 0)
    plsc.subcore_barrier()

    def body(c, _):
        pltpu.sync_copy(ones_v, acc.at[dst_v.at[c]], add=True)
        return 0

    lax.fori_loop(0, NCHUNK, body, 0)
    plsc.subcore_barrier()
    pltpu.sync_copy(acc.at[pl.ds(sid * RPT, RPT)],
                    out_hbm.at[cid, pl.ds(sid * RPT, RPT)])


@functools.partial(
    pl.kernel,
    out_type=jax.ShapeDtypeStruct((NC, N_PAD, D), jnp.float32),
    mesh=_mesh,
    scratch_types=[
        pltpu.VMEM((NCHUNK, CH), jnp.int32),
        pltpu.VMEM((NCHUNK, CH), jnp.int32),
        pltpu.VMEM((CH, D), jnp.float32),
        pltpu.VMEM_SHARED((N_PAD, D), jnp.float32),
        pltpu.SemaphoreType.DMA,
    ],
)
def _spmm_kernel(ms_hbm, src_hbm, dst_hbm, out_hbm, src_v, dst_v, rows_v, acc,
                 sem):
    cid = lax.axis_index("c")
    sid = lax.axis_index("s")
    w = cid * NS + sid

    zeros16 = jnp.zeros((16,), jnp.float32)

    def zrow(i, _):
        for j in range(D // 16):
            rows_v[i, pl.ds(j * 16, 16)] = zeros16
        return 0

    lax.fori_loop(0, CH, zrow, 0)
    pltpu.sync_copy(src_hbm.at[w], src_v)
    pltpu.sync_copy(dst_hbm.at[w], dst_v)
    for k in range(RPT // CH):
        pltpu.sync_copy(rows_v, acc.at[pl.ds(sid * RPT + k * CH, CH)])
    plsc.subcore_barrier()

    def body(c, _):
        pltpu.async_copy(ms_hbm.at[src_v.at[c]], rows_v, sem).wait()
        pltpu.sync_copy(rows_v, acc.at[dst_v.at[c]], add=True)
        return 0

    lax.fori_loop(0, NCHUNK, body, 0)
    plsc.subcore_barrier()
    pltpu.sync_copy(acc.at[pl.ds(sid * RPT, RPT)],
                    out_hbm.at[cid, pl.ds(sid * RPT, RPT)])


_RB = 1000


def _dis_from_parts(deg_ref):
    deg = deg_ref[0, :, 0] + deg_ref[1, :, 0] + 1.0
    return lax.rsqrt(deg)


def _tc1_body(x_ref, w1_ref, deg_ref, ms_ref):
    dis = _dis_from_parts(deg_ref)
    h = jnp.dot(x_ref[...], w1_ref[...], preferred_element_type=jnp.float32)
    ms_ref[...] = h * dis[:, None]


def _tc2_body(p_ref, ms1_ref, deg_ref, w2_ref, b1_ref, ms2_ref):
    dis = _dis_from_parts(deg_ref)
    s = p_ref[0] + p_ref[1] + ms1_ref[...]
    h1 = jnp.maximum(s * dis[:, None] + b1_ref[...], 0.0)
    h2 = jnp.dot(h1, w2_ref[...], preferred_element_type=jnp.float32)
    ms2_ref[...] = h2 * dis[:, None]


def _tc3_body(p_ref, ms2_ref, deg_ref, b2_ref, wl_ref, bl_ref, out_ref,
              emb_ref):
    dis = _dis_from_parts(deg_ref)
    s = p_ref[0] + p_ref[1] + ms2_ref[...]
    emb = s * dis[:, None] + b2_ref[...]
    emb_ref[...] = emb
    logits = jnp.dot(emb, wl_ref[...], preferred_element_type=jnp.float32)
    logits = logits + bl_ref[...]
    m = jnp.max(logits, axis=1, keepdims=True)
    z = logits - m
    lse = jnp.log(jnp.sum(jnp.exp(z), axis=1, keepdims=True))
    out_ref[...] = z - lse


def kernel(x, edge_index, W1, b1, W2, b2, Wl, bl):
    src = edge_index[0].reshape(NW, NCHUNK, CH)
    dst = edge_index[1].reshape(NW, NCHUNK, CH)

    deg_parts = _deg_kernel(dst)

    grid = (N // _RB,)
    full = lambda i: (0, 0)
    rowb = lambda i: (i, 0)
    degb = lambda i: (0, i, 0)
    partb = lambda i: (0, i, 0)

    deg_spec = pl.BlockSpec((NC, _RB, DEG_W), degb)
    part_spec = pl.BlockSpec((NC, _RB, D), partb)
    feat_spec = pl.BlockSpec((_RB, D), rowb)

    ms1 = pl.pallas_call(
        _tc1_body,
        grid=grid,
        in_specs=[feat_spec, pl.BlockSpec((D, D), full), deg_spec],
        out_specs=feat_spec,
        out_shape=jax.ShapeDtypeStruct((N, D), jnp.float32),
    )(x, W1, deg_parts[:, :N, :])

    p1 = _spmm_kernel(ms1, src, dst)

    ms2 = pl.pallas_call(
        _tc2_body,
        grid=grid,
        in_specs=[part_spec, feat_spec, deg_spec,
                  pl.BlockSpec((D, D), full), pl.BlockSpec((1, D), full)],
        out_specs=feat_spec,
        out_shape=jax.ShapeDtypeStruct((N, D), jnp.float32),
    )(p1[:, :N, :], ms1, deg_parts[:, :N, :], W2, b1.reshape(1, D))

    p2 = _spmm_kernel(ms2, src, dst)

    out, emb = pl.pallas_call(
        _tc3_body,
        grid=grid,
        in_specs=[part_spec, feat_spec, deg_spec,
                  pl.BlockSpec((1, D), full), pl.BlockSpec((D, OUT), full),
                  pl.BlockSpec((1, OUT), full)],
        out_specs=[pl.BlockSpec((_RB, OUT), rowb), feat_spec],
        out_shape=[jax.ShapeDtypeStruct((N, OUT), jnp.float32),
                   jax.ShapeDtypeStruct((N, D), jnp.float32)],
    )(p2[:, :N, :], ms2, deg_parts[:, :N, :], b2.reshape(1, D), Wl,
      bl.reshape(1, OUT))

    return (out, emb)


# speedup vs baseline: 23.4380x; 1.2601x over previous
import functools

import jax
import jax.numpy as jnp
from jax import lax
from jax.experimental import pallas as pl
from jax.experimental.pallas import tpu as pltpu
from jax.experimental.pallas import tpu_sc as plsc

N = 10000
E = 320000
D = 128
OUT = 40

NC = 2
NS = 16
NW = NC * NS
EPW = E // NW
CH = 80
NCHUNK = EPW // CH
N_PAD = 10240
RPT = N_PAD // NS
DEG_W = 16

_mesh = plsc.VectorSubcoreMesh(core_axis_name="c", subcore_axis_name="s")


@functools.partial(
    pl.kernel,
    out_type=jax.ShapeDtypeStruct((NC, N_PAD, DEG_W), jnp.float32),
    mesh=_mesh,
    scratch_types=[
        pltpu.VMEM((NCHUNK, CH), jnp.int32),
        pltpu.VMEM((CH, DEG_W), jnp.float32),
        pltpu.VMEM_SHARED((N_PAD, DEG_W), jnp.float32),
        pltpu.SemaphoreType.DMA,
    ],
)
def _deg_kernel(dst_hbm, out_hbm, dst_v, ones_v, acc, sem):
    cid = lax.axis_index("c")
    sid = lax.axis_index("s")
    w = cid * NS + sid

    zeros16 = jnp.zeros((16,), jnp.float32)
    ones16 = jnp.ones((16,), jnp.float32)

    def fill(i, _):
        ones_v[i, :] = ones16
        return 0

    lax.fori_loop(0, CH, fill, 0)
    pltpu.sync_copy(dst_hbm.at[w], dst_v)

    def zrow(i, _):
        ones_v[i, :] = zeros16
        return 0

    lax.fori_loop(0, CH, zrow, 0)
    for k in range(RPT // CH):
        pltpu.sync_copy(ones_v, acc.at[pl.ds(sid * RPT + k * CH, CH)])
    lax.fori_loop(0, CH, fill, 0)
    plsc.subcore_barrier()

    def body(c, _):
        pltpu.async_copy(ones_v, acc.at[dst_v.at[c]], sem, add=True)
        return 0

    lax.fori_loop(0, NCHUNK, body, 0)

    def drain(c, _):
        pltpu.make_async_copy(ones_v, acc.at[dst_v.at[c]], sem).wait()
        return 0

    lax.fori_loop(0, NCHUNK, drain, 0)
    plsc.subcore_barrier()
    pltpu.sync_copy(acc.at[pl.ds(sid * RPT, RPT)],
                    out_hbm.at[cid, pl.ds(sid * RPT, RPT)])


@functools.partial(
    pl.kernel,
    out_type=jax.ShapeDtypeStruct((NC, N_PAD, D), jnp.float32),
    mesh=_mesh,
    scratch_types=[
        pltpu.VMEM((NCHUNK, CH), jnp.int32),
        pltpu.VMEM((CH, D), jnp.float32),
        pltpu.VMEM((CH, D), jnp.float32),
        [pltpu.VMEM((1, CH), jnp.int32) for _ in range(4)],
        pltpu.VMEM_SHARED((N_PAD, D), jnp.float32),
        pltpu.SemaphoreType.DMA,
        pltpu.SemaphoreType.DMA,
        [pltpu.SemaphoreType.DMA for _ in range(4)],
    ],
)
def _spmm_kernel(ms_hbm, src_hbm, dst_hbm, out_hbm, src_v, rows0, rows1,
                 dring, acc, semr0, semr1, semi):
    cid = lax.axis_index("c")
    sid = lax.axis_index("s")
    w = cid * NS + sid

    zeros16 = jnp.zeros((16,), jnp.float32)

    def zrow(i, _):
        for j in range(D // 16):
            rows0[i, pl.ds(j * 16, 16)] = zeros16
        return 0

    lax.fori_loop(0, CH, zrow, 0)
    pltpu.sync_copy(src_hbm.at[w], src_v)
    for k in range(RPT // CH):
        pltpu.sync_copy(rows0, acc.at[pl.ds(sid * RPT + k * CH, CH)])
    plsc.subcore_barrier()

    dstw = dst_hbm.at[w]
    for j in range(4):
        pltpu.async_copy(dstw.at[pl.ds(j, 1)], dring[j], semi[j])
    pltpu.async_copy(ms_hbm.at[src_v.at[0]], rows0, semr0)

    rbufs = (rows0, rows1)
    rsems = (semr0, semr1)

    def q_body(q, _):
        c = 4 * q
        for j in range(4):
            cj = c + j
            rows, semr = rbufs[j % 2], rsems[j % 2]
            orows, osemr = rbufs[(j + 1) % 2], rsems[(j + 1) % 2]
            pltpu.make_async_copy(ms_hbm.at[src_v.at[cj]], rows, semr).wait()
            pltpu.async_copy(ms_hbm.at[src_v.at[cj + 1]], orows, osemr)
            pltpu.make_async_copy(dstw.at[pl.ds(cj, 1)], dring[j],
                                  semi[j]).wait()
            pltpu.sync_copy(rows, acc.at[dring[j].at[0]], add=True)

            @pl.when(cj + 4 < NCHUNK)
            def _():
                pltpu.async_copy(dstw.at[pl.ds(cj + 4, 1)], dring[j], semi[j])

        return 0

    lax.fori_loop(0, NCHUNK // 4, q_body, 0)
    last = NCHUNK - 1
    pltpu.make_async_copy(ms_hbm.at[src_v.at[last]], rows0, semr0).wait()
    pltpu.make_async_copy(dstw.at[pl.ds(last, 1)], dring[0], semi[0]).wait()
    pltpu.sync_copy(rows0, acc.at[dring[0].at[0]], add=True)
    plsc.subcore_barrier()
    pltpu.sync_copy(acc.at[pl.ds(sid * RPT, RPT)],
                    out_hbm.at[cid, pl.ds(sid * RPT, RPT)])


_RB = 1000


def _dis_from_parts(deg_ref):
    deg = deg_ref[0, :, 0] + deg_ref[1, :, 0] + 1.0
    return lax.rsqrt(deg)


def _tc1_body(x_ref, w1_ref, deg_ref, ms_ref):
    dis = _dis_from_parts(deg_ref)
    h = jnp.dot(x_ref[...], w1_ref[...], preferred_element_type=jnp.float32)
    ms_ref[...] = h * dis[:, None]


def _tc2_body(p_ref, ms1_ref, deg_ref, w2_ref, b1_ref, ms2_ref):
    dis = _dis_from_parts(deg_ref)
    s = p_ref[0] + p_ref[1] + ms1_ref[...]
    h1 = jnp.maximum(s * dis[:, None] + b1_ref[...], 0.0)
    h2 = jnp.dot(h1, w2_ref[...], preferred_element_type=jnp.float32)
    ms2_ref[...] = h2 * dis[:, None]


def _tc3_body(p_ref, ms2_ref, deg_ref, b2_ref, wl_ref, bl_ref, out_ref,
              emb_ref):
    dis = _dis_from_parts(deg_ref)
    s = p_ref[0] + p_ref[1] + ms2_ref[...]
    emb = s * dis[:, None] + b2_ref[...]
    emb_ref[...] = emb
    logits = jnp.dot(emb, wl_ref[...], preferred_element_type=jnp.float32)
    logits = logits + bl_ref[...]
    m = jnp.max(logits, axis=1, keepdims=True)
    z = logits - m
    lse = jnp.log(jnp.sum(jnp.exp(z), axis=1, keepdims=True))
    out_ref[...] = z - lse


def kernel(x, edge_index, W1, b1, W2, b2, Wl, bl):
    src = edge_index[0].reshape(NW, NCHUNK, CH)
    dst = edge_index[1].reshape(NW, NCHUNK, CH)

    deg_parts = _deg_kernel(dst)

    grid = (N // _RB,)
    full = lambda i: (0, 0)
    rowb = lambda i: (i, 0)
    degb = lambda i: (0, i, 0)
    partb = lambda i: (0, i, 0)

    deg_spec = pl.BlockSpec((NC, _RB, DEG_W), degb)
    part_spec = pl.BlockSpec((NC, _RB, D), partb)
    feat_spec = pl.BlockSpec((_RB, D), rowb)

    ms1 = pl.pallas_call(
        _tc1_body,
        grid=grid,
        in_specs=[feat_spec, pl.BlockSpec((D, D), full), deg_spec],
        out_specs=feat_spec,
        out_shape=jax.ShapeDtypeStruct((N, D), jnp.float32),
    )(x, W1, deg_parts[:, :N, :])

    p1 = _spmm_kernel(ms1, src, dst)

    ms2 = pl.pallas_call(
        _tc2_body,
        grid=grid,
        in_specs=[part_spec, feat_spec, deg_spec,
                  pl.BlockSpec((D, D), full), pl.BlockSpec((1, D), full)],
        out_specs=feat_spec,
        out_shape=jax.ShapeDtypeStruct((N, D), jnp.float32),
    )(p1[:, :N, :], ms1, deg_parts[:, :N, :], W2, b1.reshape(1, D))

    p2 = _spmm_kernel(ms2, src, dst)

    out, emb = pl.pallas_call(
        _tc3_body,
        grid=grid,
        in_specs=[part_spec, feat_spec, deg_spec,
                  pl.BlockSpec((1, D), full), pl.BlockSpec((D, OUT), full),
                  pl.BlockSpec((1, OUT), full)],
        out_specs=[pl.BlockSpec((_RB, OUT), rowb), feat_spec],
        out_shape=[jax.ShapeDtypeStruct((N, OUT), jnp.float32),
                   jax.ShapeDtypeStruct((N, D), jnp.float32)],
    )(p2[:, :N, :], ms2, deg_parts[:, :N, :], b2.reshape(1, D), Wl,
      bl.reshape(1, OUT))

    return (out, emb)
